# baseline (speedup 1.0000x reference)
import jax
import jax.numpy as jnp
from jax import lax
from jax.experimental import pallas as pl
from jax.experimental.pallas import tpu as pltpu
from jax.experimental.pallas import tpu_sc as plsc

B, L, D = 4096, 200, 64

_info = plsc.get_sparse_core_info()
NC, NS = _info.num_cores, _info.num_subcores
NW = NC * NS
BLK = B // NW
NL = 2
NSLAB = L // NL
LANES = 16
NG = BLK // LANES
ND8 = D // 8


def _body(idx_hbm, tok_hbm, pos_hbm, out_hbm, idx_v, pos_v, psp_v, ih0, ih1,
          rows0, rows1, wb0, wb1, gsem0, gsem1, osem):
    wid = lax.axis_index("s") * NC + lax.axis_index("c")
    b0 = wid * BLK
    pltpu.sync_copy(idx_hbm.at[:, pl.ds(b0, BLK)], idx_v)
    pltpu.sync_copy(pos_hbm, pos_v)

    rows = (rows0, rows1)
    gsems = (gsem0, gsem1)
    ihs = (ih0, ih1)
    wbs = (wb0, wb1)
    lane = lax.iota(jnp.int32, LANES)

    def start_gather(c, s):
        for lw in range(NL):
            for g in range(NG):
                sl = pl.ds(g * LANES, LANES)
                ihs[s][pl.ds(lw * BLK + g * LANES, LANES)] = (
                    idx_v[c * NL + lw, sl])
        pltpu.async_copy(tok_hbm.at[ihs[s]], rows[s], gsems[s])

    def wait_gather(c, s):
        pltpu.make_async_copy(tok_hbm.at[ihs[s]], rows[s], gsems[s]).wait()

    def start_out(c, wbuf):
        pltpu.async_copy(wbs[wbuf], out_hbm.at[pl.ds(c * NL, NL), :, wid],
                         osem)

    def wait_out(c, wbuf):
        pltpu.make_async_copy(wbs[wbuf],
                              out_hbm.at[pl.ds(c * NL, NL), :, wid],
                              osem).wait()

    def transpose_add(l, lw, buf, wb):
        for k in range(D // LANES):
            pv = pos_v[l, pl.ds(k * LANES, LANES)]
            for j in range(LANES):
                psp_v[k * LANES + j, :] = jnp.full((LANES,), pv[j])

        row_ids = [lane + (lw * BLK + g * LANES) for g in range(NG)]

        def dd_body(dd, carry):
            base = jnp.full((LANES,), dd * 8, dtype=jnp.int32)
            for j in range(8):
                d = dd * 8 + j
                p = psp_v[d, :]
                col = base + j
                for g in range(NG):
                    vals = plsc.load_gather(buf, [row_ids[g], col])
                    wb[lw, dd, j, pl.ds(g * LANES, LANES)] = vals + p
            return carry

        lax.fori_loop(0, ND8, dd_body, None)

    start_gather(0, 0)
    start_gather(1, 1)

    def group(g, carry):
        for par in range(2):
            c = 2 * g + par

            @pl.when(c >= 2)
            def _():
                wait_out(c - 2, par)

            wait_gather(c, par)
            for lw in range(NL):
                transpose_add(c * NL + lw, lw, rows[par], wbs[par])

            @pl.when(c + 2 < NSLAB)
            def _():
                start_gather(c + 2, par)

            start_out(c, par)
        return carry

    lax.fori_loop(0, NSLAB // 2, group, None)
    wait_out(NSLAB - 2, 0)
    wait_out(NSLAB - 1, 1)


def kernel(inputs, token_table, pos_table):
    out5 = pl.kernel(
        _body,
        out_type=jax.ShapeDtypeStruct((L, ND8, NW, 8, BLK), jnp.float32),
        mesh=plsc.VectorSubcoreMesh(core_axis_name="c", subcore_axis_name="s"),
        compiler_params=pltpu.CompilerParams(
            use_tc_tiling_on_sc=False, needs_layout_passes=False),
        scratch_types=[
            pltpu.VMEM((L, BLK), jnp.int32),
            pltpu.VMEM((L, D), jnp.float32),
            pltpu.VMEM((D, LANES), jnp.float32),
            pltpu.VMEM((NL * BLK,), jnp.int32),
            pltpu.VMEM((NL * BLK,), jnp.int32),
            pltpu.VMEM((NL * BLK, D), jnp.float32),
            pltpu.VMEM((NL * BLK, D), jnp.float32),
            pltpu.VMEM((NL, ND8, 8, BLK), jnp.float32),
            pltpu.VMEM((NL, ND8, 8, BLK), jnp.float32),
            pltpu.SemaphoreType.DMA,
            pltpu.SemaphoreType.DMA,
            pltpu.SemaphoreType.DMA,
        ],
    )(inputs.T, token_table, pos_table)
    return out5.transpose(2, 4, 0, 1, 3).reshape(B, L, D)


# speedup vs baseline: 1.5901x; 1.5901x over previous
import jax
import jax.numpy as jnp
from jax import lax
from jax.experimental import pallas as pl
from jax.experimental.pallas import tpu as pltpu
from jax.experimental.pallas import tpu_sc as plsc

B, L, D = 4096, 200, 64
N_ROWS = B * L

_info = plsc.get_sparse_core_info()
NC, NS = _info.num_cores, _info.num_subcores
NW = NC * NS
ROWS_PER_W = N_ROWS // NW
SEQ_PER_CHUNK = 2
CHUNK = SEQ_PER_CHUNK * L
N_CHUNKS = ROWS_PER_W // CHUNK
LANES = 16


def _body(idx_hbm, tok_hbm, pos_hbm, out_hbm, pos_v, idx_v, rows0, rows1,
          gsem0, gsem1, osem):
    wid = lax.axis_index("s") * NC + lax.axis_index("c")
    base_w = wid * ROWS_PER_W
    pltpu.sync_copy(pos_hbm, pos_v)
    pltpu.sync_copy(idx_hbm.at[pl.ds(base_w, ROWS_PER_W)], idx_v)

    rows = (rows0, rows1)
    gsems = (gsem0, gsem1)

    def start_gather(c, b):
        pltpu.async_copy(
            tok_hbm.at[idx_v.at[pl.ds(c * CHUNK, CHUNK)]], rows[b], gsems[b])

    def wait_gather(c, b):
        pltpu.make_async_copy(
            tok_hbm.at[idx_v.at[pl.ds(c * CHUNK, CHUNK)]], rows[b],
            gsems[b]).wait()

    def start_out(c, b):
        pltpu.async_copy(
            rows[b], out_hbm.at[pl.ds(base_w + c * CHUNK, CHUNK)], osem)

    def wait_out(c, b):
        pltpu.make_async_copy(
            rows[b], out_hbm.at[pl.ds(base_w + c * CHUNK, CHUNK)],
            osem).wait()

    def add_pos(b):
        buf = rows[b]

        def add_body(r, carry):
            for s in range(SEQ_PER_CHUNK):
                for k in range(D // LANES):
                    sl = pl.ds(k * LANES, LANES)
                    buf[s * L + r, sl] = buf[s * L + r, sl] + pos_v[r, sl]
            return carry

        lax.fori_loop(0, L, add_body, None)

    start_gather(0, 0)
    start_gather(1, 1)
    wait_gather(0, 0)
    add_pos(0)
    start_out(0, 0)

    def group(g, carry):
        c1 = 2 * g + 1
        wait_out(c1 - 1, 0)
        start_gather(c1 + 1, 0)
        wait_gather(c1, 1)
        add_pos(1)
        start_out(c1, 1)

        c2 = 2 * g + 2
        wait_out(c2 - 1, 1)
        start_gather(c2 + 1, 1)
        wait_gather(c2, 0)
        add_pos(0)
        start_out(c2, 0)
        return carry

    lax.fori_loop(0, N_CHUNKS // 2 - 1, group, None)

    last = N_CHUNKS - 1
    wait_gather(last, 1)
    add_pos(1)
    start_out(last, 1)
    wait_out(last - 1, 0)
    wait_out(last, 1)


def kernel(inputs, token_table, pos_table):
    idx = inputs.reshape(-1)
    out = pl.kernel(
        _body,
        out_type=jax.ShapeDtypeStruct((N_ROWS, D), jnp.float32),
        mesh=plsc.VectorSubcoreMesh(core_axis_name="c", subcore_axis_name="s"),
        compiler_params=pltpu.CompilerParams(use_tc_tiling_on_sc=False),
        scratch_types=[
            pltpu.VMEM((L, D), jnp.float32),
            pltpu.VMEM((ROWS_PER_W,), jnp.int32),
            pltpu.VMEM((CHUNK, D), jnp.float32),
            pltpu.VMEM((CHUNK, D), jnp.float32),
            pltpu.SemaphoreType.DMA,
            pltpu.SemaphoreType.DMA,
            pltpu.SemaphoreType.DMA,
        ],
    )(idx, token_table, pos_table)
    return out.reshape(B, L, D)
